# initial kernel scaffold (unmeasured)
import jax
import jax.numpy as jnp
from jax import lax
from jax.experimental import pallas as pl
from jax.experimental.pallas import tpu as pltpu

N_DEV = 4
M_PER = 1024
K = 4096
N = 2048
N_PER = N // N_DEV


def kernel(x, w_mat, scale_x, scale_w):
    def body(x_ref, w_ref, sx_ref, sw_ref, out_ref,
             ybuf, recv_buf, send_sems, recv_sems):
        my = lax.axis_index("i")

        barrier_sem = pltpu.get_barrier_semaphore()
        for off in range(1, N_DEV):
            peer = lax.rem(my + off, N_DEV)
            pl.semaphore_signal(barrier_sem, inc=1, device_id=(peer,),
                                device_id_type=pl.DeviceIdType.MESH)
        pl.semaphore_wait(barrier_sem, N_DEV - 1)

        xb = x_ref[...].astype(jnp.bfloat16)
        wb = w_ref[...].astype(jnp.bfloat16)
        acc = jnp.dot(xb, wb, preferred_element_type=jnp.float32)
        s = sx_ref[0] * sw_ref[0]
        v = acc * s
        y = v * jax.nn.sigmoid(v)
        ybuf[...] = y.astype(jnp.bfloat16)

        rdmas = []
        for h in range(1, N_DEV):
            dst = lax.rem(my + h, N_DEV)
            rdma = pltpu.make_async_remote_copy(
                src_ref=ybuf.at[:, pl.ds(dst * N_PER, N_PER)],
                dst_ref=recv_buf.at[h],
                send_sem=send_sems.at[h],
                recv_sem=recv_sems.at[h],
                device_id=(dst,),
                device_id_type=pl.DeviceIdType.MESH,
            )
            rdma.start()
            rdmas.append(rdma)

        out_ref[pl.ds(my * M_PER, M_PER), :] = (
            ybuf[:, pl.ds(my * N_PER, N_PER)].astype(jnp.float32))

        for h in range(1, N_DEV):
            src = lax.rem(my - h + N_DEV, N_DEV)
            rdmas[h - 1].wait_recv()
            out_ref[pl.ds(src * M_PER, M_PER), :] = (
                recv_buf[h].astype(jnp.float32))

        for rdma in rdmas:
            rdma.wait_send()

    return pl.pallas_call(
        body,
        out_shape=jax.ShapeDtypeStruct((N_DEV * M_PER, N_PER), jnp.float32),
        in_specs=[
            pl.BlockSpec(memory_space=pltpu.VMEM),
            pl.BlockSpec(memory_space=pltpu.VMEM),
            pl.BlockSpec(memory_space=pltpu.SMEM),
            pl.BlockSpec(memory_space=pltpu.SMEM),
        ],
        out_specs=pl.BlockSpec(memory_space=pltpu.VMEM),
        scratch_shapes=[
            pltpu.VMEM((M_PER, N), jnp.bfloat16),
            pltpu.VMEM((N_DEV, M_PER, N_PER), jnp.bfloat16),
            pltpu.SemaphoreType.DMA((N_DEV,)),
            pltpu.SemaphoreType.DMA((N_DEV,)),
        ],
        compiler_params=pltpu.CompilerParams(collective_id=0),
    )(x, w_mat, scale_x, scale_w)


# baseline (device time: 51518 ns/iter reference)
import jax
import jax.numpy as jnp
from jax import lax
from jax.experimental import pallas as pl
from jax.experimental.pallas import tpu as pltpu

N_DEV = 4
M_PER = 1024
K = 4096
N = 2048
N_PER = N // N_DEV


def kernel(x, w_mat, scale_x, scale_w):
    xb = x.astype(jnp.bfloat16)

    def body(x_ref, w_ref, sx_ref, sw_ref, out_ref,
             send_buf, recv_buf, send_sems, recv_sems):
        h = pl.program_id(0)
        my = lax.axis_index("i")

        @pl.when(h == 0)
        def _():
            barrier_sem = pltpu.get_barrier_semaphore()
            for off in range(1, N_DEV):
                peer = lax.rem(my + off, N_DEV)
                pl.semaphore_signal(barrier_sem, inc=1, device_id=(peer,),
                                    device_id_type=pl.DeviceIdType.MESH)
            pl.semaphore_wait(barrier_sem, N_DEV - 1)

        acc = jnp.dot(x_ref[...], w_ref[...].astype(jnp.bfloat16),
                      preferred_element_type=jnp.float32)
        s = sx_ref[0] * sw_ref[0]
        v = acc * s
        y = v * jax.nn.sigmoid(v)

        def desc(slot, off):
            return pltpu.make_async_remote_copy(
                src_ref=send_buf.at[slot],
                dst_ref=recv_buf.at[off],
                send_sem=send_sems.at[slot],
                recv_sem=recv_sems.at[off],
                device_id=(lax.rem(my + off, N_DEV),),
                device_id_type=pl.DeviceIdType.MESH,
            )

        for hh in range(N_DEV - 1):
            @pl.when(h == hh)
            def _(hh=hh):
                send_buf[hh] = y.astype(jnp.bfloat16)
                desc(hh, hh + 1).start()

        @pl.when(h == N_DEV - 1)
        def _():
            out_ref[pl.ds(my * M_PER, M_PER), :] = y
            for off in range(1, N_DEV):
                src = lax.rem(my - off + N_DEV, N_DEV)
                desc(0, off).wait_recv()
                out_ref[pl.ds(src * M_PER, M_PER), :] = (
                    recv_buf[off].astype(jnp.float32))
            for slot in range(N_DEV - 1):
                desc(slot, 1).wait_send()

    return pl.pallas_call(
        body,
        grid=(N_DEV,),
        out_shape=jax.ShapeDtypeStruct((N_DEV * M_PER, N_PER), jnp.float32),
        in_specs=[
            pl.BlockSpec((M_PER, K), lambda h: (0, 0),
                         memory_space=pltpu.VMEM),
            pl.BlockSpec((K, N_PER),
                         lambda h: (0, lax.rem(lax.axis_index("i") + 1 + h,
                                               N_DEV)),
                         memory_space=pltpu.VMEM),
            pl.BlockSpec(memory_space=pltpu.SMEM),
            pl.BlockSpec(memory_space=pltpu.SMEM),
        ],
        out_specs=pl.BlockSpec((N_DEV * M_PER, N_PER), lambda h: (0, 0),
                               memory_space=pltpu.VMEM),
        scratch_shapes=[
            pltpu.VMEM((N_DEV - 1, M_PER, N_PER), jnp.bfloat16),
            pltpu.VMEM((N_DEV, M_PER, N_PER), jnp.bfloat16),
            pltpu.SemaphoreType.DMA((N_DEV - 1,)),
            pltpu.SemaphoreType.DMA((N_DEV,)),
        ],
        compiler_params=pltpu.CompilerParams(
            collective_id=0,
            dimension_semantics=("arbitrary",),
        ),
    )(xb, w_mat, scale_x, scale_w)


# device time: 41397 ns/iter; 1.2445x vs baseline; 1.2445x over previous
import jax
import jax.numpy as jnp
from jax import lax
from jax.experimental import pallas as pl
from jax.experimental.pallas import tpu as pltpu

N_DEV = 4
M_PER = 1024
K = 4096
N = 2048
N_PER = N // N_DEV
N_BLK = N_PER // 2
N_STEPS = 2 * N_DEV
N_SENDS = 2 * (N_DEV - 1)


def kernel(x, w_mat, scale_x, scale_w):
    def body(x_ref, w_ref, sx_ref, sw_ref, out_ref,
             xq, send_buf, recv_buf, ssc_buf, rsc_buf,
             send_sems, recv_sems, ssc_sems, rsc_sems):
        h = pl.program_id(0)
        my = lax.axis_index("i")

        @pl.when(h == 0)
        def _():
            barrier_sem = pltpu.get_barrier_semaphore()
            for off in range(1, N_DEV):
                peer = lax.rem(my + off, N_DEV)
                pl.semaphore_signal(barrier_sem, inc=1, device_id=(peer,),
                                    device_id_type=pl.DeviceIdType.MESH)
            pl.semaphore_wait(barrier_sem, N_DEV - 1)
            xq[...] = x_ref[...].astype(jnp.float8_e4m3fn)

        acc = jnp.dot(xq[...], w_ref[...].astype(jnp.float8_e4m3fn),
                      preferred_element_type=jnp.float32)
        s = sx_ref[0] * sw_ref[0]
        v = acc * s
        y = v * jax.nn.sigmoid(v)

        def peer_of(q):
            return lax.rem(my + q // 2 + 1, N_DEV)

        def data_desc(q):
            return pltpu.make_async_remote_copy(
                src_ref=send_buf.at[q], dst_ref=recv_buf.at[q],
                send_sem=send_sems.at[q], recv_sem=recv_sems.at[q],
                device_id=(peer_of(q),),
                device_id_type=pl.DeviceIdType.MESH,
            )

        def scale_desc(q):
            return pltpu.make_async_remote_copy(
                src_ref=ssc_buf.at[q], dst_ref=rsc_buf.at[q],
                send_sem=ssc_sems.at[q], recv_sem=rsc_sems.at[q],
                device_id=(peer_of(q),),
                device_id_type=pl.DeviceIdType.MESH,
            )

        for q in range(N_SENDS):
            @pl.when(h == q)
            def _(q=q):
                m = jnp.maximum(jnp.max(jnp.abs(y)), 1e-20)
                send_buf[q] = jnp.clip(
                    jnp.round(y * (127.0 / m)), -127.0, 127.0
                ).astype(jnp.int8)
                ssc_buf[q] = jnp.full((8, 128), m * (1.0 / 127.0),
                                      jnp.float32)
                data_desc(q).start()
                scale_desc(q).start()

        for q in range(N_SENDS, N_STEPS):
            @pl.when(h == q)
            def _(q=q):
                half = q % 2
                out_ref[pl.ds(my * M_PER, M_PER),
                        pl.ds(half * N_BLK, N_BLK)] = y

        @pl.when(h == N_STEPS - 1)
        def _():
            for q in range(N_SENDS):
                off = q // 2 + 1
                half = q % 2
                src = lax.rem(my - off + N_DEV, N_DEV)
                scale_desc(q).wait_recv()
                data_desc(q).wait_recv()
                sc = jnp.max(rsc_buf[q])
                out_ref[pl.ds(src * M_PER, M_PER),
                        pl.ds(half * N_BLK, N_BLK)] = (
                    recv_buf[q].astype(jnp.float32) * sc)
            for q in range(N_SENDS):
                data_desc(q).wait_send()
                scale_desc(q).wait_send()

    return pl.pallas_call(
        body,
        grid=(N_STEPS,),
        out_shape=jax.ShapeDtypeStruct((N_DEV * M_PER, N_PER), jnp.float32),
        in_specs=[
            pl.BlockSpec((M_PER, K), lambda h: (0, 0),
                         memory_space=pltpu.VMEM),
            pl.BlockSpec(
                (K, N_BLK),
                lambda h: (0, 2 * lax.rem(lax.axis_index("i") + h // 2 + 1,
                                          N_DEV) + h % 2),
                memory_space=pltpu.VMEM),
            pl.BlockSpec(memory_space=pltpu.SMEM),
            pl.BlockSpec(memory_space=pltpu.SMEM),
        ],
        out_specs=pl.BlockSpec((N_DEV * M_PER, N_PER), lambda h: (0, 0),
                               memory_space=pltpu.VMEM),
        scratch_shapes=[
            pltpu.VMEM((M_PER, K), jnp.float8_e4m3fn),
            pltpu.VMEM((N_SENDS, M_PER, N_BLK), jnp.int8),
            pltpu.VMEM((N_SENDS, M_PER, N_BLK), jnp.int8),
            pltpu.VMEM((N_SENDS, 8, 128), jnp.float32),
            pltpu.VMEM((N_SENDS, 8, 128), jnp.float32),
            pltpu.SemaphoreType.DMA((N_SENDS,)),
            pltpu.SemaphoreType.DMA((N_SENDS,)),
            pltpu.SemaphoreType.DMA((N_SENDS,)),
            pltpu.SemaphoreType.DMA((N_SENDS,)),
        ],
        compiler_params=pltpu.CompilerParams(
            collective_id=0,
            dimension_semantics=("arbitrary",),
            vmem_limit_bytes=100 * 1024 * 1024,
        ),
    )(x, w_mat, scale_x, scale_w)
